# baseline (speedup 1.0000x reference)
import functools

import jax
import jax.numpy as jnp
from jax.experimental import pallas as pl
from jax.experimental.pallas import tpu as pltpu


def _meta_body(nexp, tg, n_units, x_ref, wr_ref,
               pos_ref, starts_ref, ut_ref, ue_ref, fi_ref, va_ref,
               wab_ref):
    f32 = jnp.float32
    xt = x_ref[...].astype(jnp.bfloat16)
    logits = jnp.dot(xt, wr_ref[...].astype(jnp.bfloat16),
                     preferred_element_type=f32)
    scores = jax.nn.sigmoid(logits)
    iota = jax.lax.broadcasted_iota(jnp.int32, scores.shape, 1)
    i1 = jnp.argmax(scores, axis=1)[:, None]
    s1 = jnp.max(scores, axis=1)[:, None]
    masked = jnp.where(iota == i1, -1.0, scores)
    i2 = jnp.argmax(masked, axis=1)[:, None]
    s2 = jnp.max(masked, axis=1)[:, None]
    denom = s1 + s2 + 1e-20
    wab_ref[...] = jnp.concatenate([s1, s2], axis=1) / denom

    bl = 128
    nb = 2 * scores.shape[0] // bl
    erow = jnp.arange(nexp, dtype=jnp.int32)[None, :]
    r_i = jax.lax.broadcasted_iota(jnp.int32, (bl, bl), 0)
    c_i = jax.lax.broadcasted_iota(jnp.int32, (bl, bl), 1)
    tril = (c_i < r_i).astype(jnp.bfloat16)
    ohs, withins, bsums = [], [], []
    for bk in range(nb):
        src = i1 if bk < nb // 2 else i2
        off = (bk % (nb // 2)) * bl
        blk = jax.lax.slice(src, (off, 0), (off + bl, 1))
        ohb = (blk == erow).astype(f32)
        withins.append(jnp.dot(tril, ohb.astype(jnp.bfloat16),
                               preferred_element_type=f32))
        bsums.append(jnp.sum(ohb, axis=0, keepdims=True))
        ohs.append(ohb)
    bsum = jnp.concatenate(bsums, axis=0)
    rb_i = jax.lax.broadcasted_iota(jnp.int32, (nb, nb), 0)
    cb_i = jax.lax.broadcasted_iota(jnp.int32, (nb, nb), 1)
    trilb = (cb_i < rb_i).astype(jnp.bfloat16)
    boff = jnp.dot(trilb, bsum.astype(jnp.bfloat16),
                   preferred_element_type=f32)
    counts = jnp.sum(bsum, axis=0, keepdims=True)
    parts = [jnp.zeros((1, 1), f32)]
    run = jnp.zeros((1, 1), f32)
    for e in range(nexp - 1):
        run = run + jax.lax.slice(counts, (0, e), (1, e + 1))
        parts.append(run)
    start = jnp.concatenate(parts, axis=1)
    total = run + jax.lax.slice(counts, (0, nexp - 1), (1, nexp))

    pos_parts = []
    for bk in range(nb):
        ohb = ohs[bk]
        rank = jnp.sum((withins[bk] + boff[bk:bk + 1, :]) * ohb,
                       axis=1, keepdims=True)
        base = jnp.sum(start * ohb, axis=1, keepdims=True)
        pos_parts.append(rank + base)
    pos = jnp.concatenate(pos_parts, axis=0)
    pos_ref[...] = pos.astype(jnp.int32)
    starts_ref[...] = jnp.concatenate([start, total], axis=1
                                      ).astype(jnp.int32)

    nt = 2 * scores.shape[0] // tg
    pfirst = (jax.lax.broadcasted_iota(jnp.int32, (nt, 1), 0)
              .astype(f32) * tg)
    plast = pfirst + (tg - 1)
    fe = jnp.sum((pfirst >= start).astype(f32), axis=1, keepdims=True) - 1
    le = jnp.sum((plast >= start).astype(f32), axis=1, keepdims=True) - 1
    cnt = le - fe + 1
    ti = jax.lax.broadcasted_iota(jnp.int32, (nt, nt), 0)
    tj = jax.lax.broadcasted_iota(jnp.int32, (nt, nt), 1)
    trilt = (tj <= ti).astype(jnp.bfloat16)
    cum_inc = jnp.dot(trilt, cnt.astype(jnp.bfloat16),
                      preferred_element_type=f32)
    cum_exc = cum_inc - cnt
    u_total = jax.lax.slice(cum_inc, (nt - 1, 0), (nt, 1))

    uu = jax.lax.broadcasted_iota(jnp.int32, (n_units, 1), 0).astype(f32)
    cum_inc_r = jnp.transpose(cum_inc)
    cum_exc_r = jnp.transpose(cum_exc)
    fe_r = jnp.transpose(fe)
    unit_tile = jnp.clip(
        jnp.sum((uu >= cum_inc_r).astype(f32), axis=1, keepdims=True),
        0.0, float(nt - 1))
    trow = jax.lax.broadcasted_iota(jnp.int32, (n_units, nt), 1).astype(f32)
    oh_t = (unit_tile == trow).astype(f32)
    fe_u = jnp.sum(oh_t * fe_r, axis=1, keepdims=True)
    ce_u = jnp.sum(oh_t * cum_exc_r, axis=1, keepdims=True)
    valid = (uu < u_total)
    unit_expert = jnp.clip(fe_u + uu - ce_u, 0.0, float(nexp - 1))
    is_first = jnp.logical_and(uu == ce_u, valid)
    ut_ref[...] = unit_tile.astype(jnp.int32)
    ue_ref[...] = unit_expert.astype(jnp.int32)
    fi_ref[...] = is_first.astype(jnp.int32)
    va_ref[...] = valid.astype(jnp.int32)


def _gather_body(t, pos_ref, xin_ref, xout_ref):
    def body(j, carry):
        row = xin_ref[pl.ds(j, 1)]
        xout_ref[pl.ds(pos_ref[j], 1)] = row
        xout_ref[pl.ds(pos_ref[j + t], 1)] = row
        return carry

    jax.lax.fori_loop(0, t, body, 0)


def _grouped_body(tg, ut_ref, ue_ref, fi_ref, va_ref, st_ref,
                  xs_ref, w1_ref, w3_ref, w2_ref, o_ref):
    u = pl.program_id(0)
    xt = xs_ref[...].astype(jnp.bfloat16)
    w1 = w1_ref[0].astype(jnp.bfloat16)
    w3 = w3_ref[0].astype(jnp.bfloat16)
    w2 = w2_ref[0].astype(jnp.bfloat16)
    a = jnp.dot(xt, w1, preferred_element_type=jnp.float32)
    bm = jnp.dot(xt, w3, preferred_element_type=jnp.float32)
    h = (a * jax.nn.sigmoid(a) * bm).astype(jnp.bfloat16)
    y = jnp.dot(h, w2, preferred_element_type=jnp.float32)
    ue = ue_ref[u, 0]
    p = (jax.lax.broadcasted_iota(jnp.int32, (y.shape[0], 1), 0)
         + ut_ref[u, 0] * tg)
    ok = jnp.logical_and(
        jnp.logical_and(p >= st_ref[0, ue], p < st_ref[0, ue + 1]),
        va_ref[u, 0] == 1)
    y = y * jnp.where(ok, 1.0, 0.0)

    @pl.when(fi_ref[u, 0] == 1)
    def _init():
        o_ref[...] = y

    @pl.when(fi_ref[u, 0] == 0)
    def _acc():
        o_ref[...] = o_ref[...] + y


def _combine_body(tc, t, d, pos_ref,
                  xb_ref, ws1_ref, ws3_ref, ws2_ref, os_ref, wab_ref,
                  o_ref, ga_ref, gb_ref):
    i = pl.program_id(0)
    base = i * tc

    def body(j, carry):
        ga_ref[pl.ds(j, 1)] = os_ref[pl.ds(pos_ref[base + j], 1)]
        gb_ref[pl.ds(j, 1)] = os_ref[pl.ds(pos_ref[t + base + j], 1)]
        return carry

    jax.lax.fori_loop(0, tc, body, 0)

    xt = xb_ref[...].astype(jnp.bfloat16)
    ws1 = ws1_ref[...].astype(jnp.bfloat16)
    ws3 = ws3_ref[...].astype(jnp.bfloat16)
    ws2 = ws2_ref[...].astype(jnp.bfloat16)
    a = jnp.dot(xt, ws1, preferred_element_type=jnp.float32)
    bm = jnp.dot(xt, ws3, preferred_element_type=jnp.float32)
    h = (a * jax.nn.sigmoid(a) * bm).astype(jnp.bfloat16)
    y = jnp.dot(h, ws2, preferred_element_type=jnp.float32)

    wab = wab_ref[...]
    wa = wab[:, 0:1, None]
    wb = wab[:, 1:2, None]
    comb = (ga_ref[...] * wa + gb_ref[...] * wb).reshape(tc, d)
    o_ref[...] = y + comb


def kernel(x, Wr, W1, W3, W2, Ws1, Ws3, Ws2):
    b, s, d = x.shape
    t = b * s
    nexp, _, dff = W1.shape
    sdff = Ws1.shape[1]
    na = 2 * t
    dsub = d // 128
    flat = x.reshape(t, d)

    tg = min(128, na)
    nt = na // tg
    n_units = nt + nexp - 1

    pos, starts, ut, ue, fi, va, wab = pl.pallas_call(
        functools.partial(_meta_body, nexp, tg, n_units),
        grid=(1,),
        in_specs=[
            pl.BlockSpec((t, d), lambda i: (0, 0)),
            pl.BlockSpec((d, nexp), lambda i: (0, 0)),
        ],
        out_specs=[
            pl.BlockSpec((na, 1), lambda i: (0, 0)),
            pl.BlockSpec((1, nexp + 1), lambda i: (0, 0)),
            pl.BlockSpec((n_units, 1), lambda i: (0, 0)),
            pl.BlockSpec((n_units, 1), lambda i: (0, 0)),
            pl.BlockSpec((n_units, 1), lambda i: (0, 0)),
            pl.BlockSpec((n_units, 1), lambda i: (0, 0)),
            pl.BlockSpec((t, 2), lambda i: (0, 0)),
        ],
        out_shape=[
            jax.ShapeDtypeStruct((na, 1), jnp.int32),
            jax.ShapeDtypeStruct((1, nexp + 1), jnp.int32),
            jax.ShapeDtypeStruct((n_units, 1), jnp.int32),
            jax.ShapeDtypeStruct((n_units, 1), jnp.int32),
            jax.ShapeDtypeStruct((n_units, 1), jnp.int32),
            jax.ShapeDtypeStruct((n_units, 1), jnp.int32),
            jax.ShapeDtypeStruct((t, 2), jnp.float32),
        ],
    )(flat, Wr)
    pos1 = pos.reshape(na)

    x_sorted3 = pl.pallas_call(
        functools.partial(_gather_body, t),
        grid_spec=pltpu.PrefetchScalarGridSpec(
            num_scalar_prefetch=1,
            grid=(1,),
            in_specs=[pl.BlockSpec((t, dsub, 128), lambda i, pr: (0, 0, 0))],
            out_specs=pl.BlockSpec((na, dsub, 128), lambda i, pr: (0, 0, 0)),
        ),
        out_shape=jax.ShapeDtypeStruct((na, dsub, 128), jnp.float32),
    )(pos1, flat.reshape(t, dsub, 128))

    out_sorted = pl.pallas_call(
        functools.partial(_grouped_body, tg),
        grid_spec=pltpu.PrefetchScalarGridSpec(
            num_scalar_prefetch=5,
            grid=(n_units,),
            in_specs=[
                pl.BlockSpec((tg, d), lambda u, ut, ue, fi, va, st: (ut[u, 0], 0)),
                pl.BlockSpec((1, d, dff), lambda u, ut, ue, fi, va, st: (ue[u, 0], 0, 0)),
                pl.BlockSpec((1, d, dff), lambda u, ut, ue, fi, va, st: (ue[u, 0], 0, 0)),
                pl.BlockSpec((1, dff, d), lambda u, ut, ue, fi, va, st: (ue[u, 0], 0, 0)),
            ],
            out_specs=pl.BlockSpec((tg, d), lambda u, ut, ue, fi, va, st: (ut[u, 0], 0)),
        ),
        out_shape=jax.ShapeDtypeStruct((na, d), jnp.float32),
    )(ut, ue, fi, va, starts, x_sorted3.reshape(na, d), W1, W3, W2)

    tc = min(128, t)
    out = pl.pallas_call(
        functools.partial(_combine_body, tc, t, d),
        grid_spec=pltpu.PrefetchScalarGridSpec(
            num_scalar_prefetch=1,
            grid=(t // tc,),
            in_specs=[
                pl.BlockSpec((tc, d), lambda i, pr: (i, 0)),
                pl.BlockSpec((d, sdff), lambda i, pr: (0, 0)),
                pl.BlockSpec((d, sdff), lambda i, pr: (0, 0)),
                pl.BlockSpec((sdff, d), lambda i, pr: (0, 0)),
                pl.BlockSpec((na, dsub, 128), lambda i, pr: (0, 0, 0)),
                pl.BlockSpec((tc, 2), lambda i, pr: (i, 0)),
            ],
            out_specs=pl.BlockSpec((tc, d), lambda i, pr: (i, 0)),
            scratch_shapes=[
                pltpu.VMEM((tc, dsub, 128), jnp.float32),
                pltpu.VMEM((tc, dsub, 128), jnp.float32),
            ],
        ),
        out_shape=jax.ShapeDtypeStruct((t, d), jnp.float32),
    )(pos1, flat, Ws1, Ws3, Ws2, out_sorted.reshape(na, dsub, 128), wab)

    return out.reshape(b, s, d)


# speedup vs baseline: 1.0182x; 1.0182x over previous
import functools

import jax
import jax.numpy as jnp
from jax.experimental import pallas as pl
from jax.experimental.pallas import tpu as pltpu


def _meta_body(nexp, tg, n_units, x_ref, wr_ref,
               pos_ref, starts_ref, ut_ref, ue_ref, fi_ref, va_ref,
               wab_ref):
    f32 = jnp.float32
    xt = x_ref[...].astype(jnp.bfloat16)
    logits = jnp.dot(xt, wr_ref[...].astype(jnp.bfloat16),
                     preferred_element_type=f32)
    scores = jax.nn.sigmoid(logits)
    iota = jax.lax.broadcasted_iota(jnp.int32, scores.shape, 1)
    i1 = jnp.argmax(scores, axis=1)[:, None]
    s1 = jnp.max(scores, axis=1)[:, None]
    masked = jnp.where(iota == i1, -1.0, scores)
    i2 = jnp.argmax(masked, axis=1)[:, None]
    s2 = jnp.max(masked, axis=1)[:, None]
    denom = s1 + s2 + 1e-20
    wab_ref[...] = jnp.concatenate([s1, s2], axis=1) / denom

    bl = 128
    nb = 2 * scores.shape[0] // bl
    erow = jnp.arange(nexp, dtype=jnp.int32)[None, :]
    r_i = jax.lax.broadcasted_iota(jnp.int32, (bl, bl), 0)
    c_i = jax.lax.broadcasted_iota(jnp.int32, (bl, bl), 1)
    tril = (c_i < r_i).astype(jnp.bfloat16)
    ohs, withins, bsums = [], [], []
    for bk in range(nb):
        src = i1 if bk < nb // 2 else i2
        off = (bk % (nb // 2)) * bl
        blk = jax.lax.slice(src, (off, 0), (off + bl, 1))
        ohb = (blk == erow).astype(f32)
        withins.append(jnp.dot(tril, ohb.astype(jnp.bfloat16),
                               preferred_element_type=f32))
        bsums.append(jnp.sum(ohb, axis=0, keepdims=True))
        ohs.append(ohb)
    bsum = jnp.concatenate(bsums, axis=0)
    rb_i = jax.lax.broadcasted_iota(jnp.int32, (nb, nb), 0)
    cb_i = jax.lax.broadcasted_iota(jnp.int32, (nb, nb), 1)
    trilb = (cb_i < rb_i).astype(jnp.bfloat16)
    boff = jnp.dot(trilb, bsum.astype(jnp.bfloat16),
                   preferred_element_type=f32)
    counts = jnp.sum(bsum, axis=0, keepdims=True)
    parts = [jnp.zeros((1, 1), f32)]
    run = jnp.zeros((1, 1), f32)
    for e in range(nexp - 1):
        run = run + jax.lax.slice(counts, (0, e), (1, e + 1))
        parts.append(run)
    start = jnp.concatenate(parts, axis=1)
    total = run + jax.lax.slice(counts, (0, nexp - 1), (1, nexp))

    pos_parts = []
    for bk in range(nb):
        ohb = ohs[bk]
        rank = jnp.sum((withins[bk] + boff[bk:bk + 1, :]) * ohb,
                       axis=1, keepdims=True)
        base = jnp.sum(start * ohb, axis=1, keepdims=True)
        pos_parts.append(rank + base)
    pos = jnp.concatenate(pos_parts, axis=0)
    pos_ref[...] = pos.astype(jnp.int32)
    starts_ref[...] = jnp.concatenate([start, total], axis=1
                                      ).astype(jnp.int32)

    nt = 2 * scores.shape[0] // tg
    pfirst = (jax.lax.broadcasted_iota(jnp.int32, (nt, 1), 0)
              .astype(f32) * tg)
    plast = pfirst + (tg - 1)
    fe = jnp.sum((pfirst >= start).astype(f32), axis=1, keepdims=True) - 1
    le = jnp.sum((plast >= start).astype(f32), axis=1, keepdims=True) - 1
    cnt = le - fe + 1
    ti = jax.lax.broadcasted_iota(jnp.int32, (nt, nt), 0)
    tj = jax.lax.broadcasted_iota(jnp.int32, (nt, nt), 1)
    trilt = (tj <= ti).astype(jnp.bfloat16)
    cum_inc = jnp.dot(trilt, cnt.astype(jnp.bfloat16),
                      preferred_element_type=f32)
    cum_exc = cum_inc - cnt
    u_total = jax.lax.slice(cum_inc, (nt - 1, 0), (nt, 1))

    uu = jax.lax.broadcasted_iota(jnp.int32, (n_units, 1), 0).astype(f32)
    cum_inc_r = jnp.transpose(cum_inc)
    cum_exc_r = jnp.transpose(cum_exc)
    fe_r = jnp.transpose(fe)
    unit_tile = jnp.clip(
        jnp.sum((uu >= cum_inc_r).astype(f32), axis=1, keepdims=True),
        0.0, float(nt - 1))
    trow = jax.lax.broadcasted_iota(jnp.int32, (n_units, nt), 1).astype(f32)
    oh_t = (unit_tile == trow).astype(f32)
    fe_u = jnp.sum(oh_t * fe_r, axis=1, keepdims=True)
    ce_u = jnp.sum(oh_t * cum_exc_r, axis=1, keepdims=True)
    valid = (uu < u_total)
    unit_expert = jnp.clip(fe_u + uu - ce_u, 0.0, float(nexp - 1))
    is_first = jnp.logical_and(uu == ce_u, valid)
    ut_ref[...] = unit_tile.astype(jnp.int32)
    ue_ref[...] = unit_expert.astype(jnp.int32)
    fi_ref[...] = is_first.astype(jnp.int32)
    va_ref[...] = valid.astype(jnp.int32)


def _gather_body(t, pos_ref, xin_ref, xout_ref):
    def body(j, carry):
        row = xin_ref[pl.ds(j, 1)]
        xout_ref[pl.ds(pos_ref[j], 1)] = row
        xout_ref[pl.ds(pos_ref[j + t], 1)] = row
        return carry

    jax.lax.fori_loop(0, t, body, 0)


def _grouped_body(tg, ut_ref, ue_ref, fi_ref, va_ref, st_ref,
                  xs_ref, w1_ref, w3_ref, w2_ref, o_ref):
    u = pl.program_id(0)
    xt = xs_ref[...].astype(jnp.bfloat16)
    w1 = w1_ref[0].astype(jnp.bfloat16)
    w3 = w3_ref[0].astype(jnp.bfloat16)
    w2 = w2_ref[0].astype(jnp.bfloat16)
    a = jnp.dot(xt, w1, preferred_element_type=jnp.float32)
    bm = jnp.dot(xt, w3, preferred_element_type=jnp.float32)
    h = (a * jax.nn.sigmoid(a) * bm).astype(jnp.bfloat16)
    y = jnp.dot(h, w2, preferred_element_type=jnp.float32)
    ue = ue_ref[u, 0]
    p = (jax.lax.broadcasted_iota(jnp.int32, (y.shape[0], 1), 0)
         + ut_ref[u, 0] * tg)
    ok = jnp.logical_and(
        jnp.logical_and(p >= st_ref[0, ue], p < st_ref[0, ue + 1]),
        va_ref[u, 0] == 1)
    y = y * jnp.where(ok, 1.0, 0.0)

    @pl.when(fi_ref[u, 0] == 1)
    def _init():
        o_ref[...] = y

    @pl.when(fi_ref[u, 0] == 0)
    def _acc():
        o_ref[...] = o_ref[...] + y


def _combine_body(tc, t, d, pos_ref,
                  xb_ref, ws1_ref, ws3_ref, ws2_ref, os_ref, wab_ref,
                  o_ref, ga_ref, gb_ref):
    i = pl.program_id(0)
    base = i * tc

    def body(j, carry):
        ga_ref[pl.ds(j, 1)] = os_ref[pl.ds(pos_ref[base + j], 1)]
        gb_ref[pl.ds(j, 1)] = os_ref[pl.ds(pos_ref[t + base + j], 1)]
        return carry

    jax.lax.fori_loop(0, tc, body, 0)

    xt = xb_ref[...].astype(jnp.bfloat16)
    ws1 = ws1_ref[...].astype(jnp.bfloat16)
    ws3 = ws3_ref[...].astype(jnp.bfloat16)
    ws2 = ws2_ref[...].astype(jnp.bfloat16)
    a = jnp.dot(xt, ws1, preferred_element_type=jnp.float32)
    bm = jnp.dot(xt, ws3, preferred_element_type=jnp.float32)
    h = (a * jax.nn.sigmoid(a) * bm).astype(jnp.bfloat16)
    y = jnp.dot(h, ws2, preferred_element_type=jnp.float32)

    wab = wab_ref[...]
    wa = wab[:, 0:1, None]
    wb = wab[:, 1:2, None]
    comb = (ga_ref[...] * wa + gb_ref[...] * wb).reshape(tc, d)
    o_ref[...] = y + comb


def kernel(x, Wr, W1, W3, W2, Ws1, Ws3, Ws2):
    b, s, d = x.shape
    t = b * s
    nexp, _, dff = W1.shape
    sdff = Ws1.shape[1]
    na = 2 * t
    dsub = d // 128
    flat = x.reshape(t, d)

    tg = min(256, na)
    nt = na // tg
    n_units = nt + nexp - 1

    pos, starts, ut, ue, fi, va, wab = pl.pallas_call(
        functools.partial(_meta_body, nexp, tg, n_units),
        grid=(1,),
        in_specs=[
            pl.BlockSpec((t, d), lambda i: (0, 0)),
            pl.BlockSpec((d, nexp), lambda i: (0, 0)),
        ],
        out_specs=[
            pl.BlockSpec((na, 1), lambda i: (0, 0)),
            pl.BlockSpec((1, nexp + 1), lambda i: (0, 0)),
            pl.BlockSpec((n_units, 1), lambda i: (0, 0)),
            pl.BlockSpec((n_units, 1), lambda i: (0, 0)),
            pl.BlockSpec((n_units, 1), lambda i: (0, 0)),
            pl.BlockSpec((n_units, 1), lambda i: (0, 0)),
            pl.BlockSpec((t, 2), lambda i: (0, 0)),
        ],
        out_shape=[
            jax.ShapeDtypeStruct((na, 1), jnp.int32),
            jax.ShapeDtypeStruct((1, nexp + 1), jnp.int32),
            jax.ShapeDtypeStruct((n_units, 1), jnp.int32),
            jax.ShapeDtypeStruct((n_units, 1), jnp.int32),
            jax.ShapeDtypeStruct((n_units, 1), jnp.int32),
            jax.ShapeDtypeStruct((n_units, 1), jnp.int32),
            jax.ShapeDtypeStruct((t, 2), jnp.float32),
        ],
    )(flat, Wr)
    pos1 = pos.reshape(na)

    x_sorted3 = pl.pallas_call(
        functools.partial(_gather_body, t),
        grid_spec=pltpu.PrefetchScalarGridSpec(
            num_scalar_prefetch=1,
            grid=(1,),
            in_specs=[pl.BlockSpec((t, dsub, 128), lambda i, pr: (0, 0, 0))],
            out_specs=pl.BlockSpec((na, dsub, 128), lambda i, pr: (0, 0, 0)),
        ),
        out_shape=jax.ShapeDtypeStruct((na, dsub, 128), jnp.float32),
    )(pos1, flat.reshape(t, dsub, 128))

    out_sorted = pl.pallas_call(
        functools.partial(_grouped_body, tg),
        grid_spec=pltpu.PrefetchScalarGridSpec(
            num_scalar_prefetch=5,
            grid=(n_units,),
            in_specs=[
                pl.BlockSpec((tg, d), lambda u, ut, ue, fi, va, st: (ut[u, 0], 0)),
                pl.BlockSpec((1, d, dff), lambda u, ut, ue, fi, va, st: (ue[u, 0], 0, 0)),
                pl.BlockSpec((1, d, dff), lambda u, ut, ue, fi, va, st: (ue[u, 0], 0, 0)),
                pl.BlockSpec((1, dff, d), lambda u, ut, ue, fi, va, st: (ue[u, 0], 0, 0)),
            ],
            out_specs=pl.BlockSpec((tg, d), lambda u, ut, ue, fi, va, st: (ut[u, 0], 0)),
        ),
        out_shape=jax.ShapeDtypeStruct((na, d), jnp.float32),
    )(ut, ue, fi, va, starts, x_sorted3.reshape(na, d), W1, W3, W2)

    tc = min(128, t)
    out = pl.pallas_call(
        functools.partial(_combine_body, tc, t, d),
        grid_spec=pltpu.PrefetchScalarGridSpec(
            num_scalar_prefetch=1,
            grid=(t // tc,),
            in_specs=[
                pl.BlockSpec((tc, d), lambda i, pr: (i, 0)),
                pl.BlockSpec((d, sdff), lambda i, pr: (0, 0)),
                pl.BlockSpec((d, sdff), lambda i, pr: (0, 0)),
                pl.BlockSpec((sdff, d), lambda i, pr: (0, 0)),
                pl.BlockSpec((na, dsub, 128), lambda i, pr: (0, 0, 0)),
                pl.BlockSpec((tc, 2), lambda i, pr: (i, 0)),
            ],
            out_specs=pl.BlockSpec((tc, d), lambda i, pr: (i, 0)),
            scratch_shapes=[
                pltpu.VMEM((tc, dsub, 128), jnp.float32),
                pltpu.VMEM((tc, dsub, 128), jnp.float32),
            ],
        ),
        out_shape=jax.ShapeDtypeStruct((t, d), jnp.float32),
    )(pos1, flat, Ws1, Ws3, Ws2, out_sorted.reshape(na, dsub, 128), wab)

    return out.reshape(b, s, d)


# speedup vs baseline: 1.1319x; 1.1117x over previous
import functools

import jax
import jax.numpy as jnp
from jax import lax
from jax.experimental import pallas as pl
from jax.experimental.pallas import tpu as pltpu
from jax.experimental.pallas import tpu_sc as plsc


def _meta_body(nexp, tg, n_units, x_ref, wr_ref,
               pos_ref, starts_ref, ut_ref, ue_ref, fi_ref, va_ref,
               wab_ref):
    f32 = jnp.float32
    xt = x_ref[...].astype(jnp.bfloat16)
    logits = jnp.dot(xt, wr_ref[...].astype(jnp.bfloat16),
                     preferred_element_type=f32)
    scores = jax.nn.sigmoid(logits)
    iota = jax.lax.broadcasted_iota(jnp.int32, scores.shape, 1)
    i1 = jnp.argmax(scores, axis=1)[:, None]
    s1 = jnp.max(scores, axis=1)[:, None]
    masked = jnp.where(iota == i1, -1.0, scores)
    i2 = jnp.argmax(masked, axis=1)[:, None]
    s2 = jnp.max(masked, axis=1)[:, None]
    denom = s1 + s2 + 1e-20
    wab_ref[...] = jnp.concatenate([s1, s2], axis=1) / denom

    bl = 128
    nb = 2 * scores.shape[0] // bl
    erow = jnp.arange(nexp, dtype=jnp.int32)[None, :]
    r_i = jax.lax.broadcasted_iota(jnp.int32, (bl, bl), 0)
    c_i = jax.lax.broadcasted_iota(jnp.int32, (bl, bl), 1)
    tril = (c_i < r_i).astype(jnp.bfloat16)
    ohs, withins, bsums = [], [], []
    for bk in range(nb):
        src = i1 if bk < nb // 2 else i2
        off = (bk % (nb // 2)) * bl
        blk = jax.lax.slice(src, (off, 0), (off + bl, 1))
        ohb = (blk == erow).astype(f32)
        withins.append(jnp.dot(tril, ohb.astype(jnp.bfloat16),
                               preferred_element_type=f32))
        bsums.append(jnp.sum(ohb, axis=0, keepdims=True))
        ohs.append(ohb)
    bsum = jnp.concatenate(bsums, axis=0)
    rb_i = jax.lax.broadcasted_iota(jnp.int32, (nb, nb), 0)
    cb_i = jax.lax.broadcasted_iota(jnp.int32, (nb, nb), 1)
    trilb = (cb_i < rb_i).astype(jnp.bfloat16)
    boff = jnp.dot(trilb, bsum.astype(jnp.bfloat16),
                   preferred_element_type=f32)
    counts = jnp.sum(bsum, axis=0, keepdims=True)
    parts = [jnp.zeros((1, 1), f32)]
    run = jnp.zeros((1, 1), f32)
    for e in range(nexp - 1):
        run = run + jax.lax.slice(counts, (0, e), (1, e + 1))
        parts.append(run)
    start = jnp.concatenate(parts, axis=1)
    total = run + jax.lax.slice(counts, (0, nexp - 1), (1, nexp))

    pos_parts = []
    for bk in range(nb):
        ohb = ohs[bk]
        rank = jnp.sum((withins[bk] + boff[bk:bk + 1, :]) * ohb,
                       axis=1, keepdims=True)
        base = jnp.sum(start * ohb, axis=1, keepdims=True)
        pos_parts.append(rank + base)
    pos = jnp.concatenate(pos_parts, axis=0)
    pos_ref[...] = pos.astype(jnp.int32)
    starts_ref[...] = jnp.concatenate([start, total], axis=1
                                      ).astype(jnp.int32)

    nt = 2 * scores.shape[0] // tg
    pfirst = (jax.lax.broadcasted_iota(jnp.int32, (nt, 1), 0)
              .astype(f32) * tg)
    plast = pfirst + (tg - 1)
    fe = jnp.sum((pfirst >= start).astype(f32), axis=1, keepdims=True) - 1
    le = jnp.sum((plast >= start).astype(f32), axis=1, keepdims=True) - 1
    cnt = le - fe + 1
    ti = jax.lax.broadcasted_iota(jnp.int32, (nt, nt), 0)
    tj = jax.lax.broadcasted_iota(jnp.int32, (nt, nt), 1)
    trilt = (tj <= ti).astype(jnp.bfloat16)
    cum_inc = jnp.dot(trilt, cnt.astype(jnp.bfloat16),
                      preferred_element_type=f32)
    cum_exc = cum_inc - cnt
    u_total = jax.lax.slice(cum_inc, (nt - 1, 0), (nt, 1))

    uu = jax.lax.broadcasted_iota(jnp.int32, (n_units, 1), 0).astype(f32)
    cum_inc_r = jnp.transpose(cum_inc)
    cum_exc_r = jnp.transpose(cum_exc)
    fe_r = jnp.transpose(fe)
    unit_tile = jnp.clip(
        jnp.sum((uu >= cum_inc_r).astype(f32), axis=1, keepdims=True),
        0.0, float(nt - 1))
    trow = jax.lax.broadcasted_iota(jnp.int32, (n_units, nt), 1).astype(f32)
    oh_t = (unit_tile == trow).astype(f32)
    fe_u = jnp.sum(oh_t * fe_r, axis=1, keepdims=True)
    ce_u = jnp.sum(oh_t * cum_exc_r, axis=1, keepdims=True)
    valid = (uu < u_total)
    unit_expert = jnp.clip(fe_u + uu - ce_u, 0.0, float(nexp - 1))
    is_first = jnp.logical_and(uu == ce_u, valid)
    ut_ref[...] = unit_tile.astype(jnp.int32)
    ue_ref[...] = unit_expert.astype(jnp.int32)
    fi_ref[...] = is_first.astype(jnp.int32)
    va_ref[...] = valid.astype(jnp.int32)


def _sc_permute(t, na, d, flat, pos1):
    info = plsc.get_sparse_core_info()
    nc, ns = info.num_cores, info.num_subcores
    nw = nc * ns
    jpw = na // nw
    ch = min(64, jpw)
    mesh = plsc.VectorSubcoreMesh(core_axis_name="c", subcore_axis_name="s")

    @functools.partial(
        pl.kernel, mesh=mesh,
        out_type=jax.ShapeDtypeStruct((na, d), jnp.float32),
        scratch_types=[
            pltpu.VMEM((ch,), jnp.int32),
            pltpu.VMEM((ch, d), jnp.float32),
        ],
    )
    def k(flat_hbm, pos_hbm, xs_hbm, idx_v, rows_v):
        wid = lax.axis_index("s") * nc + lax.axis_index("c")
        base = wid * jpw
        for cch in range(jpw // ch):
            off = base + cch * ch
            pltpu.sync_copy(pos_hbm.at[pl.ds(off, ch)], idx_v)
            pltpu.sync_copy(flat_hbm.at[pl.ds(lax.rem(off, t), ch)], rows_v)
            pltpu.sync_copy(rows_v, xs_hbm.at[idx_v])

    return k(flat, pos1)


def _grouped_body(tg, ut_ref, ue_ref, fi_ref, va_ref, st_ref,
                  xs_ref, w1_ref, w3_ref, w2_ref, o_ref):
    u = pl.program_id(0)
    xt = xs_ref[...].astype(jnp.bfloat16)
    w1 = w1_ref[0].astype(jnp.bfloat16)
    w3 = w3_ref[0].astype(jnp.bfloat16)
    w2 = w2_ref[0].astype(jnp.bfloat16)
    a = jnp.dot(xt, w1, preferred_element_type=jnp.float32)
    bm = jnp.dot(xt, w3, preferred_element_type=jnp.float32)
    h = (a * jax.nn.sigmoid(a) * bm).astype(jnp.bfloat16)
    y = jnp.dot(h, w2, preferred_element_type=jnp.float32)
    ue = ue_ref[u, 0]
    p = (jax.lax.broadcasted_iota(jnp.int32, (y.shape[0], 1), 0)
         + ut_ref[u, 0] * tg)
    ok = jnp.logical_and(
        jnp.logical_and(p >= st_ref[0, ue], p < st_ref[0, ue + 1]),
        va_ref[u, 0] == 1)
    y = y * jnp.where(ok, 1.0, 0.0)

    @pl.when(fi_ref[u, 0] == 1)
    def _init():
        o_ref[...] = y

    @pl.when(fi_ref[u, 0] == 0)
    def _acc():
        o_ref[...] = o_ref[...] + y


def _combine_body(tc, t, d, pos_ref,
                  xb_ref, ws1_ref, ws3_ref, ws2_ref, os_ref, wab_ref,
                  o_ref, ga_ref, gb_ref):
    i = pl.program_id(0)
    base = i * tc

    def body(j, carry):
        ga_ref[pl.ds(j, 1)] = os_ref[pl.ds(pos_ref[base + j], 1)]
        gb_ref[pl.ds(j, 1)] = os_ref[pl.ds(pos_ref[t + base + j], 1)]
        return carry

    jax.lax.fori_loop(0, tc, body, 0)

    xt = xb_ref[...].astype(jnp.bfloat16)
    ws1 = ws1_ref[...].astype(jnp.bfloat16)
    ws3 = ws3_ref[...].astype(jnp.bfloat16)
    ws2 = ws2_ref[...].astype(jnp.bfloat16)
    a = jnp.dot(xt, ws1, preferred_element_type=jnp.float32)
    bm = jnp.dot(xt, ws3, preferred_element_type=jnp.float32)
    h = (a * jax.nn.sigmoid(a) * bm).astype(jnp.bfloat16)
    y = jnp.dot(h, ws2, preferred_element_type=jnp.float32)

    wab = wab_ref[...]
    wa = wab[:, 0:1, None]
    wb = wab[:, 1:2, None]
    comb = (ga_ref[...] * wa + gb_ref[...] * wb).reshape(tc, d)
    o_ref[...] = y + comb


def kernel(x, Wr, W1, W3, W2, Ws1, Ws3, Ws2):
    b, s, d = x.shape
    t = b * s
    nexp, _, dff = W1.shape
    sdff = Ws1.shape[1]
    na = 2 * t
    dsub = d // 128
    flat = x.reshape(t, d)

    tg = min(256, na)
    nt = na // tg
    n_units = nt + nexp - 1

    pos, starts, ut, ue, fi, va, wab = pl.pallas_call(
        functools.partial(_meta_body, nexp, tg, n_units),
        grid=(1,),
        in_specs=[
            pl.BlockSpec((t, d), lambda i: (0, 0)),
            pl.BlockSpec((d, nexp), lambda i: (0, 0)),
        ],
        out_specs=[
            pl.BlockSpec((na, 1), lambda i: (0, 0)),
            pl.BlockSpec((1, nexp + 1), lambda i: (0, 0)),
            pl.BlockSpec((n_units, 1), lambda i: (0, 0)),
            pl.BlockSpec((n_units, 1), lambda i: (0, 0)),
            pl.BlockSpec((n_units, 1), lambda i: (0, 0)),
            pl.BlockSpec((n_units, 1), lambda i: (0, 0)),
            pl.BlockSpec((t, 2), lambda i: (0, 0)),
        ],
        out_shape=[
            jax.ShapeDtypeStruct((na, 1), jnp.int32),
            jax.ShapeDtypeStruct((1, nexp + 1), jnp.int32),
            jax.ShapeDtypeStruct((n_units, 1), jnp.int32),
            jax.ShapeDtypeStruct((n_units, 1), jnp.int32),
            jax.ShapeDtypeStruct((n_units, 1), jnp.int32),
            jax.ShapeDtypeStruct((n_units, 1), jnp.int32),
            jax.ShapeDtypeStruct((t, 2), jnp.float32),
        ],
    )(flat, Wr)
    pos1 = pos.reshape(na)

    x_sorted = _sc_permute(t, na, d, flat, pos1)

    out_sorted = pl.pallas_call(
        functools.partial(_grouped_body, tg),
        grid_spec=pltpu.PrefetchScalarGridSpec(
            num_scalar_prefetch=5,
            grid=(n_units,),
            in_specs=[
                pl.BlockSpec((tg, d), lambda u, ut, ue, fi, va, st: (ut[u, 0], 0)),
                pl.BlockSpec((1, d, dff), lambda u, ut, ue, fi, va, st: (ue[u, 0], 0, 0)),
                pl.BlockSpec((1, d, dff), lambda u, ut, ue, fi, va, st: (ue[u, 0], 0, 0)),
                pl.BlockSpec((1, dff, d), lambda u, ut, ue, fi, va, st: (ue[u, 0], 0, 0)),
            ],
            out_specs=pl.BlockSpec((tg, d), lambda u, ut, ue, fi, va, st: (ut[u, 0], 0)),
        ),
        out_shape=jax.ShapeDtypeStruct((na, d), jnp.float32),
    )(ut, ue, fi, va, starts, x_sorted, W1, W3, W2)

    tc = min(128, t)
    out = pl.pallas_call(
        functools.partial(_combine_body, tc, t, d),
        grid_spec=pltpu.PrefetchScalarGridSpec(
            num_scalar_prefetch=1,
            grid=(t // tc,),
            in_specs=[
                pl.BlockSpec((tc, d), lambda i, pr: (i, 0)),
                pl.BlockSpec((d, sdff), lambda i, pr: (0, 0)),
                pl.BlockSpec((d, sdff), lambda i, pr: (0, 0)),
                pl.BlockSpec((sdff, d), lambda i, pr: (0, 0)),
                pl.BlockSpec((na, dsub, 128), lambda i, pr: (0, 0, 0)),
                pl.BlockSpec((tc, 2), lambda i, pr: (i, 0)),
            ],
            out_specs=pl.BlockSpec((tc, d), lambda i, pr: (i, 0)),
            scratch_shapes=[
                pltpu.VMEM((tc, dsub, 128), jnp.float32),
                pltpu.VMEM((tc, dsub, 128), jnp.float32),
            ],
        ),
        out_shape=jax.ShapeDtypeStruct((t, d), jnp.float32),
    )(pos1, flat, Ws1, Ws3, Ws2, out_sorted.reshape(na, dsub, 128), wab)

    return out.reshape(b, s, d)


# speedup vs baseline: 1.2688x; 1.1209x over previous
import functools

import jax
import jax.numpy as jnp
from jax import lax
from jax.experimental import pallas as pl
from jax.experimental.pallas import tpu as pltpu
from jax.experimental.pallas import tpu_sc as plsc


def _meta_body(nexp, tg, n_units, x_ref, wr_ref,
               pos_ref, starts_ref, ut_ref, ue_ref, fi_ref, va_ref,
               wab_ref):
    f32 = jnp.float32
    xt = x_ref[...].astype(jnp.bfloat16)
    logits = jnp.dot(xt, wr_ref[...].astype(jnp.bfloat16),
                     preferred_element_type=f32)
    scores = jax.nn.sigmoid(logits)
    iota = jax.lax.broadcasted_iota(jnp.int32, scores.shape, 1)
    i1 = jnp.argmax(scores, axis=1)[:, None]
    s1 = jnp.max(scores, axis=1)[:, None]
    masked = jnp.where(iota == i1, -1.0, scores)
    i2 = jnp.argmax(masked, axis=1)[:, None]
    s2 = jnp.max(masked, axis=1)[:, None]
    denom = s1 + s2 + 1e-20
    wab_ref[...] = jnp.concatenate([s1, s2], axis=1) / denom

    bl = 128
    nb = 2 * scores.shape[0] // bl
    erow = jnp.arange(nexp, dtype=jnp.int32)[None, :]
    r_i = jax.lax.broadcasted_iota(jnp.int32, (bl, bl), 0)
    c_i = jax.lax.broadcasted_iota(jnp.int32, (bl, bl), 1)
    tril = (c_i < r_i).astype(jnp.bfloat16)
    ohs, withins, bsums = [], [], []
    for bk in range(nb):
        src = i1 if bk < nb // 2 else i2
        off = (bk % (nb // 2)) * bl
        blk = jax.lax.slice(src, (off, 0), (off + bl, 1))
        ohb = (blk == erow).astype(f32)
        withins.append(jnp.dot(tril, ohb.astype(jnp.bfloat16),
                               preferred_element_type=f32))
        bsums.append(jnp.sum(ohb, axis=0, keepdims=True))
        ohs.append(ohb)
    bsum = jnp.concatenate(bsums, axis=0)
    rb_i = jax.lax.broadcasted_iota(jnp.int32, (nb, nb), 0)
    cb_i = jax.lax.broadcasted_iota(jnp.int32, (nb, nb), 1)
    trilb = (cb_i < rb_i).astype(jnp.bfloat16)
    boff = jnp.dot(trilb, bsum.astype(jnp.bfloat16),
                   preferred_element_type=f32)
    counts = jnp.sum(bsum, axis=0, keepdims=True)
    parts = [jnp.zeros((1, 1), f32)]
    run = jnp.zeros((1, 1), f32)
    for e in range(nexp - 1):
        run = run + jax.lax.slice(counts, (0, e), (1, e + 1))
        parts.append(run)
    start = jnp.concatenate(parts, axis=1)
    total = run + jax.lax.slice(counts, (0, nexp - 1), (1, nexp))

    pos_parts = []
    for bk in range(nb):
        ohb = ohs[bk]
        rank = jnp.sum((withins[bk] + boff[bk:bk + 1, :]) * ohb,
                       axis=1, keepdims=True)
        base = jnp.sum(start * ohb, axis=1, keepdims=True)
        pos_parts.append(rank + base)
    pos = jnp.concatenate(pos_parts, axis=0)
    pos_ref[...] = pos.astype(jnp.int32)
    starts_ref[...] = jnp.concatenate([start, total], axis=1
                                      ).astype(jnp.int32)

    nt = 2 * scores.shape[0] // tg
    pfirst = (jax.lax.broadcasted_iota(jnp.int32, (nt, 1), 0)
              .astype(f32) * tg)
    plast = pfirst + (tg - 1)
    fe = jnp.sum((pfirst >= start).astype(f32), axis=1, keepdims=True) - 1
    le = jnp.sum((plast >= start).astype(f32), axis=1, keepdims=True) - 1
    cnt = le - fe + 1
    ti = jax.lax.broadcasted_iota(jnp.int32, (nt, nt), 0)
    tj = jax.lax.broadcasted_iota(jnp.int32, (nt, nt), 1)
    trilt = (tj <= ti).astype(jnp.bfloat16)
    cum_inc = jnp.dot(trilt, cnt.astype(jnp.bfloat16),
                      preferred_element_type=f32)
    cum_exc = cum_inc - cnt
    u_total = jax.lax.slice(cum_inc, (nt - 1, 0), (nt, 1))

    uu = jax.lax.broadcasted_iota(jnp.int32, (n_units, 1), 0).astype(f32)
    cum_inc_r = jnp.transpose(cum_inc)
    cum_exc_r = jnp.transpose(cum_exc)
    fe_r = jnp.transpose(fe)
    unit_tile = jnp.clip(
        jnp.sum((uu >= cum_inc_r).astype(f32), axis=1, keepdims=True),
        0.0, float(nt - 1))
    trow = jax.lax.broadcasted_iota(jnp.int32, (n_units, nt), 1).astype(f32)
    oh_t = (unit_tile == trow).astype(f32)
    fe_u = jnp.sum(oh_t * fe_r, axis=1, keepdims=True)
    ce_u = jnp.sum(oh_t * cum_exc_r, axis=1, keepdims=True)
    valid = (uu < u_total)
    unit_expert = jnp.clip(fe_u + uu - ce_u, 0.0, float(nexp - 1))
    is_first = jnp.logical_and(uu == ce_u, valid)
    ut_ref[...] = unit_tile.astype(jnp.int32)
    ue_ref[...] = unit_expert.astype(jnp.int32)
    fi_ref[...] = is_first.astype(jnp.int32)
    va_ref[...] = valid.astype(jnp.int32)


def _sc_permute(t, na, d, flat, pos1):
    info = plsc.get_sparse_core_info()
    nc, ns = info.num_cores, info.num_subcores
    nw = nc * ns
    jpw = na // nw
    ch = min(64, jpw)
    mesh = plsc.VectorSubcoreMesh(core_axis_name="c", subcore_axis_name="s")

    @functools.partial(
        pl.kernel, mesh=mesh,
        out_type=jax.ShapeDtypeStruct((na, d), jnp.float32),
        scratch_types=[
            pltpu.VMEM((ch,), jnp.int32),
            pltpu.VMEM((ch, d), jnp.float32),
        ],
    )
    def k(flat_hbm, pos_hbm, xs_hbm, idx_v, rows_v):
        wid = lax.axis_index("s") * nc + lax.axis_index("c")
        base = wid * jpw
        for cch in range(jpw // ch):
            off = base + cch * ch
            pltpu.sync_copy(pos_hbm.at[pl.ds(off, ch)], idx_v)
            pltpu.sync_copy(flat_hbm.at[pl.ds(lax.rem(off, t), ch)], rows_v)
            pltpu.sync_copy(rows_v, xs_hbm.at[idx_v])

    return k(flat, pos1)


def _sc_gather(na, d, os_arr, pos1):
    info = plsc.get_sparse_core_info()
    nc, ns = info.num_cores, info.num_subcores
    nw = nc * ns
    jpw = na // nw
    ch = min(64, jpw)
    mesh = plsc.VectorSubcoreMesh(core_axis_name="c", subcore_axis_name="s")

    @functools.partial(
        pl.kernel, mesh=mesh,
        out_type=jax.ShapeDtypeStruct((na, d), jnp.float32),
        scratch_types=[
            pltpu.VMEM((ch,), jnp.int32),
            pltpu.VMEM((ch, d), jnp.float32),
            pltpu.SemaphoreType.DMA,
        ],
    )
    def k(os_hbm, pos_hbm, gag_hbm, idx_v, rows_v, sem):
        wid = lax.axis_index("s") * nc + lax.axis_index("c")
        base = wid * jpw
        for cch in range(jpw // ch):
            off = base + cch * ch
            pltpu.sync_copy(pos_hbm.at[pl.ds(off, ch)], idx_v)
            pltpu.async_copy(os_hbm.at[idx_v], rows_v, sem).wait()
            pltpu.sync_copy(rows_v, gag_hbm.at[pl.ds(off, ch)])

    return k(os_arr, pos1)


def _grouped_body(tg, ut_ref, ue_ref, fi_ref, va_ref, st_ref,
                  xs_ref, w1_ref, w3_ref, w2_ref, o_ref):
    u = pl.program_id(0)
    xt = xs_ref[...].astype(jnp.bfloat16)
    w1 = w1_ref[0].astype(jnp.bfloat16)
    w3 = w3_ref[0].astype(jnp.bfloat16)
    w2 = w2_ref[0].astype(jnp.bfloat16)
    a = jnp.dot(xt, w1, preferred_element_type=jnp.float32)
    bm = jnp.dot(xt, w3, preferred_element_type=jnp.float32)
    h = (a * jax.nn.sigmoid(a) * bm).astype(jnp.bfloat16)
    y = jnp.dot(h, w2, preferred_element_type=jnp.float32)
    ue = ue_ref[u, 0]
    p = (jax.lax.broadcasted_iota(jnp.int32, (y.shape[0], 1), 0)
         + ut_ref[u, 0] * tg)
    ok = jnp.logical_and(
        jnp.logical_and(p >= st_ref[0, ue], p < st_ref[0, ue + 1]),
        va_ref[u, 0] == 1)
    y = y * jnp.where(ok, 1.0, 0.0)

    @pl.when(fi_ref[u, 0] == 1)
    def _init():
        o_ref[...] = y

    @pl.when(fi_ref[u, 0] == 0)
    def _acc():
        o_ref[...] = o_ref[...] + y


def _combine_body(xb_ref, ws1_ref, ws3_ref, ws2_ref, ga_ref, gb_ref,
                  wab_ref, o_ref):
    xt = xb_ref[...].astype(jnp.bfloat16)
    ws1 = ws1_ref[...].astype(jnp.bfloat16)
    ws3 = ws3_ref[...].astype(jnp.bfloat16)
    ws2 = ws2_ref[...].astype(jnp.bfloat16)
    a = jnp.dot(xt, ws1, preferred_element_type=jnp.float32)
    bm = jnp.dot(xt, ws3, preferred_element_type=jnp.float32)
    h = (a * jax.nn.sigmoid(a) * bm).astype(jnp.bfloat16)
    y = jnp.dot(h, ws2, preferred_element_type=jnp.float32)

    wab = wab_ref[...]
    o_ref[...] = y + ga_ref[...] * wab[:, 0:1] + gb_ref[...] * wab[:, 1:2]


def kernel(x, Wr, W1, W3, W2, Ws1, Ws3, Ws2):
    b, s, d = x.shape
    t = b * s
    nexp, _, dff = W1.shape
    sdff = Ws1.shape[1]
    na = 2 * t
    dsub = d // 128
    flat = x.reshape(t, d)

    tg = min(256, na)
    nt = na // tg
    n_units = nt + nexp - 1

    pos, starts, ut, ue, fi, va, wab = pl.pallas_call(
        functools.partial(_meta_body, nexp, tg, n_units),
        grid=(1,),
        in_specs=[
            pl.BlockSpec((t, d), lambda i: (0, 0)),
            pl.BlockSpec((d, nexp), lambda i: (0, 0)),
        ],
        out_specs=[
            pl.BlockSpec((na, 1), lambda i: (0, 0)),
            pl.BlockSpec((1, nexp + 1), lambda i: (0, 0)),
            pl.BlockSpec((n_units, 1), lambda i: (0, 0)),
            pl.BlockSpec((n_units, 1), lambda i: (0, 0)),
            pl.BlockSpec((n_units, 1), lambda i: (0, 0)),
            pl.BlockSpec((n_units, 1), lambda i: (0, 0)),
            pl.BlockSpec((t, 2), lambda i: (0, 0)),
        ],
        out_shape=[
            jax.ShapeDtypeStruct((na, 1), jnp.int32),
            jax.ShapeDtypeStruct((1, nexp + 1), jnp.int32),
            jax.ShapeDtypeStruct((n_units, 1), jnp.int32),
            jax.ShapeDtypeStruct((n_units, 1), jnp.int32),
            jax.ShapeDtypeStruct((n_units, 1), jnp.int32),
            jax.ShapeDtypeStruct((n_units, 1), jnp.int32),
            jax.ShapeDtypeStruct((t, 2), jnp.float32),
        ],
    )(flat, Wr)
    pos1 = pos.reshape(na)

    x_sorted = _sc_permute(t, na, d, flat, pos1)

    out_sorted = pl.pallas_call(
        functools.partial(_grouped_body, tg),
        grid_spec=pltpu.PrefetchScalarGridSpec(
            num_scalar_prefetch=5,
            grid=(n_units,),
            in_specs=[
                pl.BlockSpec((tg, d), lambda u, ut, ue, fi, va, st: (ut[u, 0], 0)),
                pl.BlockSpec((1, d, dff), lambda u, ut, ue, fi, va, st: (ue[u, 0], 0, 0)),
                pl.BlockSpec((1, d, dff), lambda u, ut, ue, fi, va, st: (ue[u, 0], 0, 0)),
                pl.BlockSpec((1, dff, d), lambda u, ut, ue, fi, va, st: (ue[u, 0], 0, 0)),
            ],
            out_specs=pl.BlockSpec((tg, d), lambda u, ut, ue, fi, va, st: (ut[u, 0], 0)),
        ),
        out_shape=jax.ShapeDtypeStruct((na, d), jnp.float32),
    )(ut, ue, fi, va, starts, x_sorted, W1, W3, W2)

    gag = _sc_gather(na, d, out_sorted, pos1)

    tc = min(512, t)
    ntc = t // tc
    out = pl.pallas_call(
        _combine_body,
        grid=(ntc,),
        in_specs=[
            pl.BlockSpec((tc, d), lambda i: (i, 0)),
            pl.BlockSpec((d, sdff), lambda i: (0, 0)),
            pl.BlockSpec((d, sdff), lambda i: (0, 0)),
            pl.BlockSpec((sdff, d), lambda i: (0, 0)),
            pl.BlockSpec((tc, d), lambda i: (i, 0)),
            pl.BlockSpec((tc, d), lambda i, _n=ntc: (i + _n, 0)),
            pl.BlockSpec((tc, 2), lambda i: (i, 0)),
        ],
        out_specs=pl.BlockSpec((tc, d), lambda i: (i, 0)),
        out_shape=jax.ShapeDtypeStruct((t, d), jnp.float32),
    )(flat, Ws1, Ws3, Ws2, gag, gag, wab)

    return out.reshape(b, s, d)
